# baseline (speedup 1.0000x reference)
import functools

import jax
import jax.numpy as jnp
from jax import lax
from jax.experimental import pallas as pl
from jax.experimental.pallas import tpu as pltpu
from jax.experimental.pallas import tpu_sc as plsc

_LANES = 16
_TC_BLOCK = 256


def _make_sc_partials(sc_rows, row0, feat_dim):
  info = plsc.get_sparse_core_info()
  nc, ns = info.num_cores, info.num_subcores
  nw = nc * ns
  assert sc_rows % (8 * nw) == 0
  bpw = sc_rows // nw
  ch = bpw // 2 if bpw <= 64 else 32
  assert bpw % ch == 0 and ch % 8 == 0
  nsub = bpw // ch
  groups = feat_dim // _LANES

  mesh = plsc.VectorSubcoreMesh(core_axis_name="c", subcore_axis_name="s")

  @functools.partial(
      pl.kernel,
      mesh=mesh,
      out_type=jax.ShapeDtypeStruct((nw, _LANES), jnp.float32),
      scratch_types=[
          pltpu.VMEM((bpw,), jnp.int32),
          pltpu.VMEM((ch, feat_dim), jnp.float32),
          pltpu.VMEM((ch, feat_dim), jnp.float32),
          pltpu.VMEM((ch, feat_dim), jnp.float32),
          pltpu.VMEM((ch, feat_dim), jnp.float32),
          pltpu.VMEM((_LANES,), jnp.float32),
          pltpu.SemaphoreType.DMA,
          pltpu.SemaphoreType.DMA,
          pltpu.SemaphoreType.DMA,
          pltpu.SemaphoreType.DMA,
      ],
  )
  def sc_kernel(feat_hbm, lab_hbm, cent_hbm, tca_hbm, out_hbm,
                idx_v, crows0, crows1, fb0, fb1, acc_v,
                gsem0, gsem1, fsem0, fsem1):
    wid = lax.axis_index("s") * nc + lax.axis_index("c")
    base = row0 + wid * bpw
    crows = (crows0, crows1)
    fbufs = (fb0, fb1)
    gsems = (gsem0, gsem1)
    fsems = (fsem0, fsem1)

    pltpu.sync_copy(lab_hbm.at[pl.ds(base, bpw)], idx_v)

    def issue(s):
      b = s % 2
      g = pltpu.async_copy(
          cent_hbm.at[idx_v.at[pl.ds(s * ch, ch)]], crows[b], gsems[b])
      f = pltpu.async_copy(
          feat_hbm.at[pl.ds(base + s * ch, ch)], fbufs[b], fsems[b])
      return g, f

    def accumulate(s, acc):
      b = s % 2
      fb, cb = fbufs[b], crows[b]

      def row_body(r, a):
        for j in range(groups):
          f = fb[r, pl.ds(j * _LANES, _LANES)]
          c = cb[r, pl.ds(j * _LANES, _LANES)]
          d = f - c
          a = a + d * d
        return a

      return lax.fori_loop(0, ch, row_body, acc)

    acc = jnp.zeros((_LANES,), jnp.float32)
    pltpu.sync_copy(tca_hbm.at[0], acc_v)

    pending = issue(0)
    for s in range(nsub):
      nxt = issue(s + 1) if s + 1 < nsub else None
      pending[0].wait()
      pending[1].wait()
      acc = accumulate(s, acc)
      pending = nxt

    acc = jnp.where(wid == 0, acc + acc_v[...], acc)
    acc_v[...] = acc
    pltpu.sync_copy(acc_v, out_hbm.at[wid])

  return sc_kernel, nw


def _tc_partial(features, labels3d, centers, r0, rows):
  batch, feat_dim = features.shape
  nb = rows // _TC_BLOCK
  boff = r0 // _TC_BLOCK
  num_classes = centers.shape[0]

  def body(lab_ref, f_ref, c_ref, o_ref):
    pid = pl.program_id(0)

    @pl.when(pid == 0)
    def _():
      o_ref[...] = jnp.zeros_like(o_ref)

    lab = lab_ref[0, 0, :]
    onehot = (lab[:, None] == lax.broadcasted_iota(
        jnp.int32, (_TC_BLOCK, num_classes), 1)).astype(jnp.bfloat16)
    g = jnp.dot(onehot, c_ref[...].astype(jnp.bfloat16),
                preferred_element_type=jnp.float32)
    d = f_ref[...] - g
    lane0 = lax.broadcasted_iota(jnp.int32, (1, _LANES), 1) == 0
    o_ref[...] += jnp.where(lane0, jnp.sum(d * d), 0.0)

  return pl.pallas_call(
      body,
      grid=(nb,),
      in_specs=[
          pl.BlockSpec((1, 1, _TC_BLOCK), lambda i: (i + boff, 0, 0)),
          pl.BlockSpec((_TC_BLOCK, feat_dim), lambda i: (i + boff, 0)),
          pl.BlockSpec((num_classes, feat_dim), lambda i: (0, 0)),
      ],
      out_specs=pl.BlockSpec((1, _LANES), lambda i: (0, 0)),
      out_shape=jax.ShapeDtypeStruct((1, _LANES), jnp.float32),
  )(labels3d, features, centers)


def _tc_combine(sc_partials, tcb_partial, batch):
  def body(p_ref, t_ref, o_ref):
    o_ref[0, 0] = (jnp.sum(p_ref[...]) + jnp.sum(t_ref[...])) * (0.5 / batch)

  out = pl.pallas_call(
      body,
      in_specs=[
          pl.BlockSpec(memory_space=pltpu.VMEM),
          pl.BlockSpec(memory_space=pltpu.VMEM),
      ],
      out_specs=pl.BlockSpec(memory_space=pltpu.SMEM),
      out_shape=jax.ShapeDtypeStruct((1, 1), jnp.float32),
  )(sc_partials, tcb_partial)
  return out[0, 0]


def kernel(features, labels, centers):
  batch, feat_dim = features.shape
  na = 2048
  ns_ = 1536
  labels = labels.astype(jnp.int32)
  labels3d = labels.reshape(batch // _TC_BLOCK, 1, _TC_BLOCK)

  tca = _tc_partial(features, labels3d, centers, 0, na)

  sc_kernel, nw = _make_sc_partials(ns_, na, feat_dim)
  sc_partials = sc_kernel(features, labels, centers, tca)

  tcb = _tc_partial(features, labels3d, centers, na + ns_,
                    batch - na - ns_)

  return _tc_combine(sc_partials, tcb, batch)


# speedup vs baseline: 1.2350x; 1.2350x over previous
import functools

import jax
import jax.numpy as jnp
from jax import lax
from jax.experimental import pallas as pl
from jax.experimental.pallas import tpu as pltpu
from jax.experimental.pallas import tpu_sc as plsc

_LANES = 16
_TC_BLOCK = 256


def _make_sc_partials(sc_rows, feat_dim):
  info = plsc.get_sparse_core_info()
  nc, ns = info.num_cores, info.num_subcores
  nw = nc * ns
  assert sc_rows % (8 * nw) == 0
  bpw = sc_rows // nw
  ch = bpw // 2 if bpw <= 64 else 32
  assert bpw % ch == 0 and ch % 8 == 0
  nsub = bpw // ch
  groups = feat_dim // _LANES

  mesh = plsc.VectorSubcoreMesh(core_axis_name="c", subcore_axis_name="s")

  @functools.partial(
      pl.kernel,
      mesh=mesh,
      out_type=jax.ShapeDtypeStruct((nw, _LANES), jnp.float32),
      scratch_types=[
          pltpu.VMEM((bpw,), jnp.int32),
          pltpu.VMEM((ch, feat_dim), jnp.float32),
          pltpu.VMEM((ch, feat_dim), jnp.float32),
          pltpu.VMEM((ch, feat_dim), jnp.float32),
          pltpu.VMEM((ch, feat_dim), jnp.float32),
          pltpu.VMEM((_LANES,), jnp.float32),
          pltpu.SemaphoreType.DMA,
          pltpu.SemaphoreType.DMA,
          pltpu.SemaphoreType.DMA,
          pltpu.SemaphoreType.DMA,
      ],
  )
  def sc_kernel(feat_hbm, lab_hbm, cent_hbm, out_hbm,
                idx_v, crows0, crows1, fb0, fb1, acc_v,
                gsem0, gsem1, fsem0, fsem1):
    wid = lax.axis_index("s") * nc + lax.axis_index("c")
    base = wid * bpw
    crows = (crows0, crows1)
    fbufs = (fb0, fb1)
    gsems = (gsem0, gsem1)
    fsems = (fsem0, fsem1)

    pltpu.sync_copy(lab_hbm.at[pl.ds(base, bpw)], idx_v)

    def issue(s):
      b = s % 2
      g = pltpu.async_copy(
          cent_hbm.at[idx_v.at[pl.ds(s * ch, ch)]], crows[b], gsems[b])
      f = pltpu.async_copy(
          feat_hbm.at[pl.ds(base + s * ch, ch)], fbufs[b], fsems[b])
      return g, f

    def accumulate(s, acc):
      b = s % 2
      fb, cb = fbufs[b], crows[b]

      def row_body(r, a):
        for j in range(groups):
          f = fb[r, pl.ds(j * _LANES, _LANES)]
          c = cb[r, pl.ds(j * _LANES, _LANES)]
          d = f - c
          a = a + d * d
        return a

      return lax.fori_loop(0, ch, row_body, acc)

    acc = jnp.zeros((_LANES,), jnp.float32)
    pending = issue(0)
    for s in range(nsub):
      nxt = issue(s + 1) if s + 1 < nsub else None
      pending[0].wait()
      pending[1].wait()
      acc = accumulate(s, acc)
      pending = nxt

    acc_v[...] = acc
    pltpu.sync_copy(acc_v, out_hbm.at[wid])

  return sc_kernel, nw


def _tc_partial(features, labels, centers, split):
  batch, feat_dim = features.shape
  nb = (batch - split) // _TC_BLOCK
  boff = split // _TC_BLOCK
  num_classes = centers.shape[0]

  def body(lab_ref, f_ref, c_ref, o_ref):
    pid = pl.program_id(0)

    @pl.when(pid == 0)
    def _():
      o_ref[...] = jnp.zeros_like(o_ref)

    lab = lab_ref[...]
    onehot = (lab[:, None] == lax.broadcasted_iota(
        jnp.int32, (_TC_BLOCK, num_classes), 1)).astype(jnp.bfloat16)
    g = jnp.dot(onehot, c_ref[...].astype(jnp.bfloat16),
                preferred_element_type=jnp.float32)
    d = f_ref[...] - g
    lane0 = lax.broadcasted_iota(jnp.int32, (1, _LANES), 1) == 0
    o_ref[...] += jnp.where(lane0, jnp.sum(d * d), 0.0)

  return pl.pallas_call(
      body,
      grid=(nb,),
      in_specs=[
          pl.BlockSpec((_TC_BLOCK,), lambda i: (i + boff,)),
          pl.BlockSpec((_TC_BLOCK, feat_dim), lambda i: (i + boff, 0)),
          pl.BlockSpec((num_classes, feat_dim), lambda i: (0, 0)),
      ],
      out_specs=pl.BlockSpec((1, _LANES), lambda i: (0, 0)),
      out_shape=jax.ShapeDtypeStruct((1, _LANES), jnp.float32),
  )(labels, features, centers)


def _tc_combine(sc_partials, tc_partial, batch):
  def body(p_ref, t_ref, o_ref):
    o_ref[0, 0] = (jnp.sum(p_ref[...]) + jnp.sum(t_ref[...])) * (0.5 / batch)

  out = pl.pallas_call(
      body,
      in_specs=[
          pl.BlockSpec(memory_space=pltpu.VMEM),
          pl.BlockSpec(memory_space=pltpu.VMEM),
      ],
      out_specs=pl.BlockSpec(memory_space=pltpu.SMEM),
      out_shape=jax.ShapeDtypeStruct((1, 1), jnp.float32),
  )(sc_partials, tc_partial)
  return out[0, 0]


def kernel(features, labels, centers):
  batch, feat_dim = features.shape
  split = 2048
  labels = labels.astype(jnp.int32)

  sc_kernel, nw = _make_sc_partials(split, feat_dim)
  sc_partials = sc_kernel(features, labels, centers)

  tc_part = _tc_partial(features, labels, centers, split)

  return _tc_combine(sc_partials, tc_part, batch)


# speedup vs baseline: 1.2462x; 1.0091x over previous
import functools

import jax
import jax.numpy as jnp
from jax import lax
from jax.experimental import pallas as pl
from jax.experimental.pallas import tpu as pltpu
from jax.experimental.pallas import tpu_sc as plsc

_LANES = 16
_TC_BLOCK = 512


def _make_sc_partials(sc_rows, feat_dim):
  info = plsc.get_sparse_core_info()
  nc, ns = info.num_cores, info.num_subcores
  nw = nc * ns
  assert sc_rows % (8 * nw) == 0
  bpw = sc_rows // nw
  ch = bpw // 2 if bpw <= 64 else 32
  assert bpw % ch == 0 and ch % 8 == 0
  nsub = bpw // ch
  groups = feat_dim // _LANES

  mesh = plsc.VectorSubcoreMesh(core_axis_name="c", subcore_axis_name="s")

  @functools.partial(
      pl.kernel,
      mesh=mesh,
      out_type=jax.ShapeDtypeStruct((nw, _LANES), jnp.float32),
      scratch_types=[
          pltpu.VMEM((bpw,), jnp.int32),
          pltpu.VMEM((ch, feat_dim), jnp.float32),
          pltpu.VMEM((ch, feat_dim), jnp.float32),
          pltpu.VMEM((ch, feat_dim), jnp.float32),
          pltpu.VMEM((ch, feat_dim), jnp.float32),
          pltpu.VMEM((_LANES,), jnp.float32),
          pltpu.SemaphoreType.DMA,
          pltpu.SemaphoreType.DMA,
          pltpu.SemaphoreType.DMA,
          pltpu.SemaphoreType.DMA,
      ],
  )
  def sc_kernel(feat_hbm, lab_hbm, cent_hbm, out_hbm,
                idx_v, crows0, crows1, fb0, fb1, acc_v,
                gsem0, gsem1, fsem0, fsem1):
    wid = lax.axis_index("s") * nc + lax.axis_index("c")
    base = wid * bpw
    crows = (crows0, crows1)
    fbufs = (fb0, fb1)
    gsems = (gsem0, gsem1)
    fsems = (fsem0, fsem1)

    pltpu.sync_copy(lab_hbm.at[pl.ds(base, bpw)], idx_v)

    def issue(s):
      b = s % 2
      g = pltpu.async_copy(
          cent_hbm.at[idx_v.at[pl.ds(s * ch, ch)]], crows[b], gsems[b])
      f = pltpu.async_copy(
          feat_hbm.at[pl.ds(base + s * ch, ch)], fbufs[b], fsems[b])
      return g, f

    def accumulate(s, acc):
      b = s % 2
      fb, cb = fbufs[b], crows[b]

      def row_body(r, a):
        for j in range(groups):
          f = fb[r, pl.ds(j * _LANES, _LANES)]
          c = cb[r, pl.ds(j * _LANES, _LANES)]
          d = f - c
          a = a + d * d
        return a

      return lax.fori_loop(0, ch, row_body, acc)

    acc = jnp.zeros((_LANES,), jnp.float32)
    pending = issue(0)
    for s in range(nsub):
      nxt = issue(s + 1) if s + 1 < nsub else None
      pending[0].wait()
      pending[1].wait()
      acc = accumulate(s, acc)
      pending = nxt

    acc_v[...] = acc
    pltpu.sync_copy(acc_v, out_hbm.at[wid])

  return sc_kernel, nw


def _tc_partial(features, labels, centers, split):
  batch, feat_dim = features.shape
  nb = (batch - split) // _TC_BLOCK
  boff = split // _TC_BLOCK
  num_classes = centers.shape[0]

  def body(lab_ref, f_ref, c_ref, o_ref):
    pid = pl.program_id(0)

    @pl.when(pid == 0)
    def _():
      o_ref[...] = jnp.zeros_like(o_ref)

    lab = lab_ref[...]
    onehot = (lab[:, None] == lax.broadcasted_iota(
        jnp.int32, (_TC_BLOCK, num_classes), 1)).astype(jnp.bfloat16)
    g = jnp.dot(onehot, c_ref[...].astype(jnp.bfloat16),
                preferred_element_type=jnp.float32)
    d = f_ref[...] - g
    lane0 = lax.broadcasted_iota(jnp.int32, (1, _LANES), 1) == 0
    o_ref[...] += jnp.where(lane0, jnp.sum(d * d), 0.0)

  return pl.pallas_call(
      body,
      grid=(nb,),
      in_specs=[
          pl.BlockSpec((_TC_BLOCK,), lambda i: (i + boff,)),
          pl.BlockSpec((_TC_BLOCK, feat_dim), lambda i: (i + boff, 0)),
          pl.BlockSpec((num_classes, feat_dim), lambda i: (0, 0)),
      ],
      out_specs=pl.BlockSpec((1, _LANES), lambda i: (0, 0)),
      out_shape=jax.ShapeDtypeStruct((1, _LANES), jnp.float32),
  )(labels, features, centers)


def _tc_combine(sc_partials, tc_partial, batch):
  def body(p_ref, t_ref, o_ref):
    o_ref[0, 0] = (jnp.sum(p_ref[...]) + jnp.sum(t_ref[...])) * (0.5 / batch)

  out = pl.pallas_call(
      body,
      in_specs=[
          pl.BlockSpec(memory_space=pltpu.VMEM),
          pl.BlockSpec(memory_space=pltpu.VMEM),
      ],
      out_specs=pl.BlockSpec(memory_space=pltpu.SMEM),
      out_shape=jax.ShapeDtypeStruct((1, 1), jnp.float32),
  )(sc_partials, tc_partial)
  return out[0, 0]


def kernel(features, labels, centers):
  batch, feat_dim = features.shape
  split = 2048
  labels = labels.astype(jnp.int32)

  sc_kernel, nw = _make_sc_partials(split, feat_dim)
  sc_partials = sc_kernel(features, labels, centers)

  tc_part = _tc_partial(features, labels, centers, split)

  return _tc_combine(sc_partials, tc_part, batch)


# speedup vs baseline: 1.2475x; 1.0010x over previous
import functools

import jax
import jax.numpy as jnp
from jax import lax
from jax.experimental import pallas as pl
from jax.experimental.pallas import tpu as pltpu
from jax.experimental.pallas import tpu_sc as plsc

_LANES = 16
_TC_BLOCK = 512


def _make_sc_partials(sc_rows, feat_dim, num_classes):
  del num_classes
  info = plsc.get_sparse_core_info()
  nc, ns = info.num_cores, info.num_subcores
  nw = nc * ns
  assert sc_rows % (8 * nw) == 0
  bpw = sc_rows // nw
  ch = bpw // 2 if bpw <= 64 else 32
  assert bpw % ch == 0 and ch % 8 == 0
  nsub = bpw // ch
  groups = feat_dim // _LANES

  mesh = plsc.VectorSubcoreMesh(core_axis_name="c", subcore_axis_name="s")

  @functools.partial(
      pl.kernel,
      mesh=mesh,
      out_type=jax.ShapeDtypeStruct((nw, _LANES), jnp.float32),
      scratch_types=[
          pltpu.VMEM((bpw,), jnp.int32),
          pltpu.VMEM((ch, feat_dim), jnp.float32),
          pltpu.VMEM((ch, feat_dim), jnp.float32),
          pltpu.VMEM((ch, feat_dim), jnp.float32),
          pltpu.VMEM((ch, feat_dim), jnp.float32),
          pltpu.VMEM((_LANES,), jnp.float32),
          pltpu.SemaphoreType.DMA,
          pltpu.SemaphoreType.DMA,
          pltpu.SemaphoreType.DMA,
          pltpu.SemaphoreType.DMA,
      ],
  )
  def sc_kernel(feat_hbm, lab_hbm, cent_hbm, out_hbm,
                idx_v, crows0, crows1, fb0, fb1, acc_v,
                gsem0, gsem1, fsem0, fsem1):
    wid = lax.axis_index("s") * nc + lax.axis_index("c")
    base = wid * bpw
    crows = (crows0, crows1)
    fbufs = (fb0, fb1)
    gsems = (gsem0, gsem1)
    fsems = (fsem0, fsem1)

    pltpu.sync_copy(lab_hbm.at[pl.ds(base, bpw)], idx_v)

    def issue(s):
      b = s % 2
      g = pltpu.async_copy(
          cent_hbm.at[idx_v.at[pl.ds(s * ch, ch)]], crows[b], gsems[b])
      f = pltpu.async_copy(
          feat_hbm.at[pl.ds(base + s * ch, ch)], fbufs[b], fsems[b])
      return g, f

    def accumulate(s, acc):
      b = s % 2
      fb, cb = fbufs[b], crows[b]

      def row_body(r, a):
        for j in range(groups):
          f = fb[r, pl.ds(j * _LANES, _LANES)]
          c = cb[r, pl.ds(j * _LANES, _LANES)]
          d = f - c
          a = a + d * d
        return a

      return lax.fori_loop(0, ch, row_body, acc)

    acc = jnp.zeros((_LANES,), jnp.float32)
    pending = issue(0)
    for s in range(nsub):
      nxt = issue(s + 1) if s + 1 < nsub else None
      pending[0].wait()
      pending[1].wait()
      acc = accumulate(s, acc)
      pending = nxt

    acc_v[...] = acc
    pltpu.sync_copy(acc_v, out_hbm.at[wid])

  return sc_kernel, nw


def _tc_partial(features, labels, centers, split):
  batch, feat_dim = features.shape
  nb = (batch - split) // _TC_BLOCK
  boff = split // _TC_BLOCK
  num_classes = centers.shape[0]

  def body(lab_ref, f_ref, c_ref, o_ref):
    pid = pl.program_id(0)

    @pl.when(pid == 0)
    def _():
      o_ref[...] = jnp.zeros_like(o_ref)

    lab = lab_ref[...]
    onehot = (lab[:, None] == lax.broadcasted_iota(
        jnp.int32, (_TC_BLOCK, num_classes), 1)).astype(jnp.bfloat16)
    g = jnp.dot(onehot, c_ref[...].astype(jnp.bfloat16),
                preferred_element_type=jnp.float32)
    d = f_ref[...] - g
    lane0 = lax.broadcasted_iota(jnp.int32, (1, _LANES), 1) == 0
    o_ref[...] += jnp.where(lane0, jnp.sum(d * d), 0.0)

  return pl.pallas_call(
      body,
      grid=(nb,),
      in_specs=[
          pl.BlockSpec((_TC_BLOCK,), lambda i: (i + boff,)),
          pl.BlockSpec((_TC_BLOCK, feat_dim), lambda i: (i + boff, 0)),
          pl.BlockSpec((num_classes, feat_dim), lambda i: (0, 0)),
      ],
      out_specs=pl.BlockSpec((1, _LANES), lambda i: (0, 0)),
      out_shape=jax.ShapeDtypeStruct((1, _LANES), jnp.float32),
  )(labels, features, centers)


def _tc_combine(sc_partials, tc_partial, batch):
  def body(p_ref, t_ref, o_ref):
    o_ref[0, 0] = (jnp.sum(p_ref[...]) + jnp.sum(t_ref[...])) * (0.5 / batch)

  out = pl.pallas_call(
      body,
      in_specs=[
          pl.BlockSpec(memory_space=pltpu.VMEM),
          pl.BlockSpec(memory_space=pltpu.VMEM),
      ],
      out_specs=pl.BlockSpec(memory_space=pltpu.SMEM),
      out_shape=jax.ShapeDtypeStruct((1, 1), jnp.float32),
  )(sc_partials, tc_partial)
  return out[0, 0]


def kernel(features, labels, centers):
  batch, feat_dim = features.shape
  split = 2048
  labels = labels.astype(jnp.int32)

  sc_kernel, nw = _make_sc_partials(split, feat_dim, centers.shape[0])
  sc_partials = sc_kernel(features, labels, centers)

  tc_part = _tc_partial(features, labels, centers, split)

  return _tc_combine(sc_partials, tc_part, batch)
